# baseline (speedup 1.0000x reference)
import jax
import jax.numpy as jnp
from jax import lax
from jax.experimental import pallas as pl
from jax.experimental.pallas import tpu as pltpu

_BLOCK = 640000


def _linear_kernel(x_ref, w_ref, b_ref, o_ref):
    rows = _BLOCK // 128
    xw = x_ref[...].reshape(rows, 128)
    res = lax.dot_general(
        xw, w_ref[...],
        (((1,), (0,)), ((), ())),
        preferred_element_type=jnp.float32,
    ) + b_ref[...]
    o_ref[...] = res.reshape(_BLOCK)


def kernel(x, edge_index, W, b):
    n, h = x.shape
    out_dim = W.shape[0]
    flat = n * h
    x1 = x.reshape(flat)
    wt = W.T
    wbig = jnp.zeros((2 * h, 2 * out_dim), jnp.float32)
    wbig = wbig.at[:h, :out_dim].set(wt).at[h:, out_dim:].set(wt)
    b2 = jnp.concatenate([b, b]).reshape(1, 2 * out_dim)
    out1 = pl.pallas_call(
        _linear_kernel,
        grid=(flat // _BLOCK,),
        in_specs=[
            pl.BlockSpec((_BLOCK,), lambda i: (i,)),
            pl.BlockSpec((2 * h, 2 * out_dim), lambda i: (0, 0)),
            pl.BlockSpec((1, 2 * out_dim), lambda i: (0, 0)),
        ],
        out_specs=pl.BlockSpec((_BLOCK,), lambda i: (i,)),
        out_shape=jax.ShapeDtypeStruct((flat,), jnp.float32),
        compiler_params=pltpu.CompilerParams(
            dimension_semantics=("parallel",),
        ),
    )(x1, wbig, b2)
    return out1.reshape(n, out_dim)


# speedup vs baseline: 8.0221x; 8.0221x over previous
import jax
import jax.numpy as jnp
from jax import lax
from jax.experimental import pallas as pl
from jax.experimental.pallas import tpu as pltpu

_BCOLS = 8192


def _linear_kernel(x_ref, w_ref, b_ref, o_ref):
    o_ref[...] = lax.dot_general(
        w_ref[...], x_ref[...],
        (((1,), (0,)), ((), ())),
        preferred_element_type=jnp.float32,
    ) + b_ref[...]


def kernel(x, edge_index, W, b):
    n, h = x.shape
    out_dim = W.shape[0]
    xt = x.T
    b2 = b.reshape(out_dim, 1)
    out_t = pl.pallas_call(
        _linear_kernel,
        grid=(pl.cdiv(n, _BCOLS),),
        in_specs=[
            pl.BlockSpec((h, _BCOLS), lambda i: (0, i)),
            pl.BlockSpec((out_dim, h), lambda i: (0, 0)),
            pl.BlockSpec((out_dim, 1), lambda i: (0, 0)),
        ],
        out_specs=pl.BlockSpec((out_dim, _BCOLS), lambda i: (0, i)),
        out_shape=jax.ShapeDtypeStruct((out_dim, n), jnp.float32),
        compiler_params=pltpu.CompilerParams(
            dimension_semantics=("parallel",),
        ),
    )(xt, W, b2)
    return out_t.T
